# baseline (speedup 1.0000x reference)
import functools

import jax
import jax.numpy as jnp
from jax.experimental import pallas as pl
from jax.experimental.pallas import tpu as pltpu

R = 32
V = R * R * R
NEG_SLOPE = 0.1
BN_EPS = 1e-4
CHUNK = 1024
VP = V + 1057
VP = ((VP + 7) // 8) * 8
_OFFS = (0, 1, 32, 33, 1024, 1025, 1056, 1057)


def _k0_body(coords_ref, sidx_ref, b00_ref, w8_ref):
    c = coords_ref[...]
    mean = jnp.mean(c, axis=1, keepdims=True)
    cn = c - mean
    r2 = jnp.sum(cn * cn, axis=0, keepdims=True)
    scale = jnp.sqrt(jnp.max(r2))
    f = cn / (scale * 2.0) + 0.5
    f = jnp.clip(f * R, 0.0, R - 1.0)
    v = jnp.round(f).astype(jnp.int32)
    sidx_ref[...] = (v[0:1] * R + v[1:2]) * R + v[2:3]
    lo_f = jnp.floor(f)
    lo = lo_f.astype(jnp.int32)
    b00_ref[...] = (lo[0:1] * R + lo[1:2]) * R + lo[2:3]
    fr = f - lo_f
    fx, fy, fz = fr[0:1], fr[1:2], fr[2:3]
    gx, gy, gz = 1.0 - fx, 1.0 - fy, 1.0 - fz
    w8_ref[...] = jnp.concatenate(
        [gx * gy * gz, gx * gy * fz, gx * fy * gz, gx * fy * fz,
         fx * gy * gz, fx * gy * fz, fx * fy * gz, fx * fy * fz], axis=0)


def _k1_body(nch, u, sidx_ref, fpad_ref, out_ref, acc, idx_s, sem):
    j = pl.program_id(1)

    @pl.when(j == 0)
    def _zero():
        acc[...] = jnp.zeros_like(acc)

    cp = pltpu.make_async_copy(sidx_ref, idx_s, sem)
    cp.start()
    cp.wait()

    def body(p, carry):
        for t in range(u):
            mi = p * u + t
            i = idx_s[0, 0, mi]
            acc[i, 0, :] = acc[i, 0, :] + fpad_ref[mi, 0, :]
        return carry

    jax.lax.fori_loop(0, CHUNK // u, body, 0)

    @pl.when(j == nch - 1)
    def _avg():
        for r0 in range(0, V, 2048):
            blk = acc[r0:r0 + 2048, 0, :]
            cnt = pltpu.roll(blk, 64, axis=1)
            den = jnp.maximum(cnt, 1.0)
            out_ref[r0:r0 + 2048, :] = (blk / den)[:, :R * 2]


def _k2_body(xm_ref, x0_ref, xp_ref, w_ref, a_ref, d_ref, out_ref, xz):
    a = pl.program_id(1)
    zpos = jax.lax.broadcasted_iota(jnp.int32, (CHUNK, 64), 0) % R
    zrow = jnp.zeros((1, 64), jnp.float32)
    for s, (ref, valid) in enumerate(
            ((xm_ref, a > 0), (x0_ref, a >= 0), (xp_ref, a < R - 1))):
        m = jnp.where(valid, 1.0, 0.0)
        sl = ref[...] * m
        sm = jnp.concatenate([zrow, sl[:-1]], axis=0)
        sm = jnp.where(zpos == 0, 0.0, sm)
        sp = jnp.concatenate([sl[1:], zrow], axis=0)
        sp = jnp.where(zpos == R - 1, 0.0, sp)
        base = s * 1088
        xz[base:base + 32, :] = jnp.zeros((32, 192), jnp.float32)
        xz[base + 32:base + 1056, :] = jnp.concatenate([sm, sl, sp], axis=1)
        xz[base + 1056:base + 1088, :] = jnp.zeros((32, 192), jnp.float32)

    acc = jnp.zeros((CHUNK, 64), jnp.float32)
    for s in range(3):
        for dy in range(3):
            xsl = xz[s * 1088 + dy * 32:s * 1088 + dy * 32 + CHUNK, :]
            acc = acc + jnp.dot(xsl, w_ref[s, dy],
                                preferred_element_type=jnp.float32)
    y = acc * a_ref[...] + d_ref[...]
    out_ref[...] = jnp.where(y >= 0, y, NEG_SLOPE * y)


def _k3_body(u, b00_ref, w8_ref, g_ref, out_ref, slots, idx_s, sem):
    cp = pltpu.make_async_copy(b00_ref, idx_s, sem)
    cp.start()
    cp.wait()

    def body(p, carry):
        for t in range(u):
            mi = p * u + t
            base = idx_s[0, 0, mi]
            for c in range(8):
                slots[c * CHUNK + mi, 0, :] = g_ref[base + _OFFS[c], 0, :]
        return carry

    jax.lax.fori_loop(0, CHUNK // u, body, 0)

    w = w8_ref[...]
    acc = jnp.zeros((CHUNK, 64), jnp.float32)
    for c in range(8):
        acc = acc + slots[c * CHUNK:(c + 1) * CHUNK, 0, :] * w[:, c:c + 1]
    out_ref[...] = acc


def _conv_layer(x, wc, aa, dd, b):
    return pl.pallas_call(
        _k2_body,
        grid=(b, R),
        in_specs=[
            pl.BlockSpec((None, CHUNK, 64),
                         lambda i, a: (i, jnp.maximum(a - 1, 0), 0)),
            pl.BlockSpec((None, CHUNK, 64), lambda i, a: (i, a, 0)),
            pl.BlockSpec((None, CHUNK, 64),
                         lambda i, a: (i, jnp.minimum(a + 1, R - 1), 0)),
            pl.BlockSpec((3, 3, 192, 64), lambda i, a: (0, 0, 0, 0)),
            pl.BlockSpec((1, 64), lambda i, a: (0, 0)),
            pl.BlockSpec((1, 64), lambda i, a: (0, 0)),
        ],
        out_specs=pl.BlockSpec((None, CHUNK, 64), lambda i, a: (i, a, 0)),
        out_shape=jax.ShapeDtypeStruct((b, V, 64), jnp.float32),
        scratch_shapes=[pltpu.VMEM((3 * 1088, 192), jnp.float32)],
        compiler_params=pltpu.CompilerParams(
            dimension_semantics=("parallel", "arbitrary")),
    )(x, x, x, wc, aa, dd)


def kernel(features, coords, w1, b1, g1, be1, m1, v1,
           w2, b2, g2, be2, m2, v2):
    b, c, n = features.shape
    nch = n // CHUNK

    sidx, b00, w8 = pl.pallas_call(
        _k0_body,
        grid=(b,),
        in_specs=[pl.BlockSpec((None, 3, n), lambda i: (i, 0, 0))],
        out_specs=[
            pl.BlockSpec((None, 1, n), lambda i: (i, 0, 0)),
            pl.BlockSpec((None, 1, n), lambda i: (i, 0, 0)),
            pl.BlockSpec((None, 8, n), lambda i: (i, 0, 0)),
        ],
        out_shape=[
            jax.ShapeDtypeStruct((b, 1, n), jnp.int32),
            jax.ShapeDtypeStruct((b, 1, n), jnp.int32),
            jax.ShapeDtypeStruct((b, 8, n), jnp.float32),
        ],
        compiler_params=pltpu.CompilerParams(
            dimension_semantics=("parallel",)),
    )(coords)

    feat_t = features.transpose(0, 2, 1)
    fpad = jnp.concatenate([feat_t, jnp.ones_like(feat_t)],
                           axis=-1).reshape(b, n, 1, 128)

    grid0 = pl.pallas_call(
        functools.partial(_k1_body, nch, 4),
        grid=(b, nch),
        in_specs=[
            pl.BlockSpec((None, 1, CHUNK), lambda i, j: (i, 0, j)),
            pl.BlockSpec((None, CHUNK, 1, 128), lambda i, j: (i, j, 0, 0)),
        ],
        out_specs=pl.BlockSpec((None, V, 64), lambda i, j: (i, 0, 0)),
        out_shape=jax.ShapeDtypeStruct((b, V, 64), jnp.float32),
        scratch_shapes=[
            pltpu.VMEM((V, 1, 128), jnp.float32),
            pltpu.SMEM((1, 1, CHUNK), jnp.int32),
            pltpu.SemaphoreType.DMA,
        ],
        compiler_params=pltpu.CompilerParams(
            dimension_semantics=("parallel", "arbitrary")),
    )(sidx, fpad)

    a1 = g1 / jnp.sqrt(v1 + BN_EPS)
    d1 = (b1 - m1) * a1 + be1
    a2 = g2 / jnp.sqrt(v2 + BN_EPS)
    d2 = (b2 - m2) * a2 + be2
    wc1 = w1.transpose(2, 3, 4, 1, 0).reshape(3, 3, 192, 64)
    wc2 = w2.transpose(2, 3, 4, 1, 0).reshape(3, 3, 192, 64)

    h1 = _conv_layer(grid0, wc1, a1.reshape(1, 64), d1.reshape(1, 64), b)
    h2 = _conv_layer(h1, wc2, a2.reshape(1, 64), d2.reshape(1, 64), b)

    gpad = jnp.pad(h2, ((0, 0), (0, VP - V), (0, 0))).reshape(b, VP, 1, 64)
    w8t = w8.transpose(0, 2, 1)

    out_t = pl.pallas_call(
        functools.partial(_k3_body, 4),
        grid=(b, nch),
        in_specs=[
            pl.BlockSpec((None, 1, CHUNK), lambda i, j: (i, 0, j)),
            pl.BlockSpec((None, CHUNK, 8), lambda i, j: (i, j, 0)),
            pl.BlockSpec((None, VP, 1, 64), lambda i, j: (i, 0, 0, 0)),
        ],
        out_specs=pl.BlockSpec((None, CHUNK, 64), lambda i, j: (i, j, 0)),
        out_shape=jax.ShapeDtypeStruct((b, n, 64), jnp.float32),
        scratch_shapes=[
            pltpu.VMEM((8 * CHUNK, 1, 64), jnp.float32),
            pltpu.SMEM((1, 1, CHUNK), jnp.int32),
            pltpu.SemaphoreType.DMA,
        ],
        compiler_params=pltpu.CompilerParams(
            dimension_semantics=("parallel", "arbitrary")),
    )(b00, w8t, gpad)

    return out_t.transpose(0, 2, 1), coords


# speedup vs baseline: 1.0017x; 1.0017x over previous
import functools

import jax
import jax.numpy as jnp
from jax.experimental import pallas as pl
from jax.experimental.pallas import tpu as pltpu

R = 32
V = R * R * R
NEG_SLOPE = 0.1
BN_EPS = 1e-4
CHUNK = 1024
VP = V + 1057
VP = ((VP + 7) // 8) * 8
_OFFS = (0, 1, 32, 33, 1024, 1025, 1056, 1057)


def _k0_body(coords_ref, sidx_ref, b00_ref, w8_ref):
    c = coords_ref[...]
    mean = jnp.mean(c, axis=1, keepdims=True)
    cn = c - mean
    r2 = jnp.sum(cn * cn, axis=0, keepdims=True)
    scale = jnp.sqrt(jnp.max(r2))
    f = cn / (scale * 2.0) + 0.5
    f = jnp.clip(f * R, 0.0, R - 1.0)
    v = jnp.round(f).astype(jnp.int32)
    sidx_ref[...] = (v[0:1] * R + v[1:2]) * R + v[2:3]
    lo_f = jnp.floor(f)
    lo = lo_f.astype(jnp.int32)
    b00_ref[...] = (lo[0:1] * R + lo[1:2]) * R + lo[2:3]
    fr = f - lo_f
    fx, fy, fz = fr[0:1], fr[1:2], fr[2:3]
    gx, gy, gz = 1.0 - fx, 1.0 - fy, 1.0 - fz
    w8_ref[...] = jnp.concatenate(
        [gx * gy * gz, gx * gy * fz, gx * fy * gz, gx * fy * fz,
         fx * gy * gz, fx * gy * fz, fx * fy * gz, fx * fy * fz], axis=0)


def _k1_body(nch, u, sidx_ref, fpad_ref, out_ref, acc, idx_s, sem):
    j = pl.program_id(1)

    @pl.when(j == 0)
    def _zero():
        acc[...] = jnp.zeros_like(acc)

    cp = pltpu.make_async_copy(sidx_ref, idx_s, sem)
    cp.start()
    cp.wait()

    def body(p, carry):
        for t in range(u):
            mi = p * u + t
            i = idx_s[0, 0, mi]
            acc[i, 0, :] = acc[i, 0, :] + fpad_ref[mi, 0, :]
        return carry

    jax.lax.fori_loop(0, CHUNK // u, body, 0)

    @pl.when(j == nch - 1)
    def _avg():
        for r0 in range(0, V, 2048):
            blk = acc[r0:r0 + 2048, 0, :]
            cnt = pltpu.roll(blk, 64, axis=1)
            den = jnp.maximum(cnt, 1.0)
            out_ref[r0:r0 + 2048, :] = (blk / den)[:, :R * 2]


def _k2_body(xm_ref, x0_ref, xp_ref, w_ref, a_ref, d_ref, out_ref, xz):
    a = pl.program_id(1)
    zpos = jax.lax.broadcasted_iota(jnp.int32, (CHUNK, 64), 0) % R
    zrow = jnp.zeros((1, 64), jnp.float32)
    for s, (ref, valid) in enumerate(
            ((xm_ref, a > 0), (x0_ref, a >= 0), (xp_ref, a < R - 1))):
        m = jnp.where(valid, 1.0, 0.0)
        sl = ref[...] * m
        sm = jnp.concatenate([zrow, sl[:-1]], axis=0)
        sm = jnp.where(zpos == 0, 0.0, sm)
        sp = jnp.concatenate([sl[1:], zrow], axis=0)
        sp = jnp.where(zpos == R - 1, 0.0, sp)
        base = s * 1088
        xz[base:base + 32, :] = jnp.zeros((32, 192), jnp.bfloat16)
        xz[base + 32:base + 1056, :] = jnp.concatenate(
            [sm, sl, sp], axis=1).astype(jnp.bfloat16)
        xz[base + 1056:base + 1088, :] = jnp.zeros((32, 192), jnp.bfloat16)

    acc = jnp.zeros((CHUNK, 64), jnp.float32)
    for s in range(3):
        for dy in range(3):
            xsl = xz[s * 1088 + dy * 32:s * 1088 + dy * 32 + CHUNK, :]
            acc = acc + jnp.dot(xsl, w_ref[s, dy],
                                preferred_element_type=jnp.float32)
    y = acc * a_ref[...] + d_ref[...]
    out_ref[...] = jnp.where(y >= 0, y, NEG_SLOPE * y)


def _k3_body(u, b00_ref, w8_ref, g_ref, out_ref, slots, idx_s, sem):
    cp = pltpu.make_async_copy(b00_ref, idx_s, sem)
    cp.start()
    cp.wait()

    def body(p, carry):
        for t in range(u):
            mi = p * u + t
            base = idx_s[0, 0, mi]
            for c in range(8):
                slots[c * CHUNK + mi, 0, :] = g_ref[base + _OFFS[c], 0, :]
        return carry

    jax.lax.fori_loop(0, CHUNK // u, body, 0)

    w = w8_ref[...]
    acc = jnp.zeros((CHUNK, 64), jnp.float32)
    for c in range(8):
        acc = acc + slots[c * CHUNK:(c + 1) * CHUNK, 0, :] * w[:, c:c + 1]
    out_ref[...] = acc


def _conv_layer(x, wc, aa, dd, b):
    return pl.pallas_call(
        _k2_body,
        grid=(b, R),
        in_specs=[
            pl.BlockSpec((None, CHUNK, 64),
                         lambda i, a: (i, jnp.maximum(a - 1, 0), 0)),
            pl.BlockSpec((None, CHUNK, 64), lambda i, a: (i, a, 0)),
            pl.BlockSpec((None, CHUNK, 64),
                         lambda i, a: (i, jnp.minimum(a + 1, R - 1), 0)),
            pl.BlockSpec((3, 3, 192, 64), lambda i, a: (0, 0, 0, 0)),
            pl.BlockSpec((1, 64), lambda i, a: (0, 0)),
            pl.BlockSpec((1, 64), lambda i, a: (0, 0)),
        ],
        out_specs=pl.BlockSpec((None, CHUNK, 64), lambda i, a: (i, a, 0)),
        out_shape=jax.ShapeDtypeStruct((b, V, 64), jnp.float32),
        scratch_shapes=[pltpu.VMEM((3 * 1088, 192), jnp.bfloat16)],
        compiler_params=pltpu.CompilerParams(
            dimension_semantics=("parallel", "arbitrary")),
    )(x, x, x, wc, aa, dd)


def kernel(features, coords, w1, b1, g1, be1, m1, v1,
           w2, b2, g2, be2, m2, v2):
    b, c, n = features.shape
    nch = n // CHUNK

    sidx, b00, w8 = pl.pallas_call(
        _k0_body,
        grid=(b,),
        in_specs=[pl.BlockSpec((None, 3, n), lambda i: (i, 0, 0))],
        out_specs=[
            pl.BlockSpec((None, 1, n), lambda i: (i, 0, 0)),
            pl.BlockSpec((None, 1, n), lambda i: (i, 0, 0)),
            pl.BlockSpec((None, 8, n), lambda i: (i, 0, 0)),
        ],
        out_shape=[
            jax.ShapeDtypeStruct((b, 1, n), jnp.int32),
            jax.ShapeDtypeStruct((b, 1, n), jnp.int32),
            jax.ShapeDtypeStruct((b, 8, n), jnp.float32),
        ],
        compiler_params=pltpu.CompilerParams(
            dimension_semantics=("parallel",)),
    )(coords)

    feat_t = features.transpose(0, 2, 1)
    fpad = jnp.concatenate([feat_t, jnp.ones_like(feat_t)],
                           axis=-1).reshape(b, n, 1, 128)

    grid0 = pl.pallas_call(
        functools.partial(_k1_body, nch, 4),
        grid=(b, nch),
        in_specs=[
            pl.BlockSpec((None, 1, CHUNK), lambda i, j: (i, 0, j)),
            pl.BlockSpec((None, CHUNK, 1, 128), lambda i, j: (i, j, 0, 0)),
        ],
        out_specs=pl.BlockSpec((None, V, 64), lambda i, j: (i, 0, 0)),
        out_shape=jax.ShapeDtypeStruct((b, V, 64), jnp.float32),
        scratch_shapes=[
            pltpu.VMEM((V, 1, 128), jnp.float32),
            pltpu.SMEM((1, 1, CHUNK), jnp.int32),
            pltpu.SemaphoreType.DMA,
        ],
        compiler_params=pltpu.CompilerParams(
            dimension_semantics=("parallel", "arbitrary")),
    )(sidx, fpad)

    a1 = g1 / jnp.sqrt(v1 + BN_EPS)
    d1 = (b1 - m1) * a1 + be1
    a2 = g2 / jnp.sqrt(v2 + BN_EPS)
    d2 = (b2 - m2) * a2 + be2
    wc1 = w1.transpose(2, 3, 4, 1, 0).reshape(3, 3, 192, 64)
    wc2 = w2.transpose(2, 3, 4, 1, 0).reshape(3, 3, 192, 64)

    h1 = _conv_layer(grid0, wc1.astype(jnp.bfloat16), a1.reshape(1, 64), d1.reshape(1, 64), b)
    h2 = _conv_layer(h1, wc2.astype(jnp.bfloat16), a2.reshape(1, 64), d2.reshape(1, 64), b)

    gpad = jnp.pad(h2, ((0, 0), (0, VP - V), (0, 0))).reshape(b, VP, 1, 64)
    w8t = w8.transpose(0, 2, 1)

    out_t = pl.pallas_call(
        functools.partial(_k3_body, 4),
        grid=(b, nch),
        in_specs=[
            pl.BlockSpec((None, 1, CHUNK), lambda i, j: (i, 0, j)),
            pl.BlockSpec((None, CHUNK, 8), lambda i, j: (i, j, 0)),
            pl.BlockSpec((None, VP, 1, 64), lambda i, j: (i, 0, 0, 0)),
        ],
        out_specs=pl.BlockSpec((None, CHUNK, 64), lambda i, j: (i, j, 0)),
        out_shape=jax.ShapeDtypeStruct((b, n, 64), jnp.float32),
        scratch_shapes=[
            pltpu.VMEM((8 * CHUNK, 1, 64), jnp.float32),
            pltpu.SMEM((1, 1, CHUNK), jnp.int32),
            pltpu.SemaphoreType.DMA,
        ],
        compiler_params=pltpu.CompilerParams(
            dimension_semantics=("parallel", "arbitrary")),
    )(b00, w8t, gpad)

    return out_t.transpose(0, 2, 1), coords
